# Initial kernel scaffold; baseline (speedup 1.0000x reference)
import jax
import jax.numpy as jnp
from jax.experimental import pallas as pl


def kernel(x, edge_index, batch, edge_attr, W1, b1, W2, b2, W3, b3, Wl, bl):
    raise NotImplementedError("write your pallas kernel here")


# speedup vs baseline: 9.3101x; 9.3101x over previous
import dataclasses
import functools

import jax
import jax.numpy as jnp
from jax import lax
from jax.experimental import pallas as pl
from jax.experimental.pallas import tpu as pltpu
from jax.experimental.pallas import tpu_sc as plsc

N = 10000
E = 320000
D = 128
G = 128

NC = 2
NS = 16
NW = NC * NS
L = 16

CK = 128
NCHUNK = E // CK
BASE_CH = NCHUNK // NW
EXTRA = NCHUNK - BASE_CH * NW

ROWS_A = 624
ROWS_LAST = N - 15 * ROWS_A

_mesh = plsc.VectorSubcoreMesh(core_axis_name="c", subcore_axis_name="s")

_sc_params = pltpu.CompilerParams()
if "needs_layout_passes" in pltpu.CompilerParams.__dataclass_fields__:
    _sc_params = dataclasses.replace(_sc_params, needs_layout_passes=False)


def _edge_loop(wid, process):
    @pl.loop(0, BASE_CH)
    def _(i):
        process((i * NW + wid) * CK)

    @pl.when(wid < EXTRA)
    def _():
        process((BASE_CH * NW + wid) * CK)


def _tile_rows(s, fn):
    @pl.when(s < NS - 1)
    def _():
        fn(s * ROWS_A, ROWS_A)

    @pl.when(s == NS - 1)
    def _():
        fn((NS - 1) * ROWS_A, ROWS_LAST)


@functools.partial(
    pl.kernel,
    out_type=jax.ShapeDtypeStruct((NC, N, D), jnp.float32),
    mesh=_mesh,
    compiler_params=_sc_params,
    scratch_types=[
        pltpu.VMEM_SHARED((N, D), jnp.float32),
        pltpu.VMEM((CK,), jnp.int32),
        pltpu.VMEM((1, CK), jnp.int32),
        pltpu.VMEM((CK,), jnp.float32),
        pltpu.VMEM((CK, D), jnp.float32),
        pltpu.SemaphoreType.DMA,
    ],
)
def _spmm_sc(y_hbm, row_hbm, col_hbm, ew_hbm, zero_hbm, out_hbm,
             acc, rowb, colb, ewb, gb, sem):
    c = lax.axis_index("c")
    s = lax.axis_index("s")
    wid = s * NC + c

    _tile_rows(s, lambda base, n: pltpu.sync_copy(
        zero_hbm.at[pl.ds(base, n)], acc.at[pl.ds(base, n)]))
    plsc.subcore_barrier()

    def process(off):
        pltpu.sync_copy(row_hbm.at[pl.ds(off, CK)], rowb)
        pltpu.sync_copy(col_hbm.at[pl.ds(off, CK)], colb.at[0])
        pltpu.sync_copy(ew_hbm.at[pl.ds(off, CK)], ewb)
        pltpu.async_copy(y_hbm.at[rowb], gb, sem).wait()

        @pl.loop(0, CK)
        def _(k):
            wv = plsc.load_gather(ewb, [jnp.full((L,), k, jnp.int32)])
            for j in range(D // L):
                sl = pl.ds(j * L, L)
                gb[k, sl] = gb[k, sl] * wv

        pltpu.sync_copy(gb, acc.at[colb.at[0]], add=True)

    _edge_loop(wid, process)
    plsc.subcore_barrier()

    _tile_rows(s, lambda base, n: pltpu.sync_copy(
        acc.at[pl.ds(base, n)], out_hbm.at[c, pl.ds(base, n)]))


@functools.partial(
    pl.kernel,
    out_type=jax.ShapeDtypeStruct((NW, N), jnp.float32),
    mesh=_mesh,
    compiler_params=_sc_params,
    scratch_types=[
        pltpu.VMEM((N,), jnp.float32),
        pltpu.VMEM((CK,), jnp.int32),
        pltpu.VMEM((CK,), jnp.float32),
    ],
)
def _deg_sc(col_hbm, ew_hbm, out_hbm, acc, colb, ewb):
    c = lax.axis_index("c")
    s = lax.axis_index("s")
    wid = s * NC + c

    @pl.loop(0, N // L)
    def _(i):
        acc[pl.ds(i * L, L)] = jnp.zeros((L,), jnp.float32)

    def process(off):
        pltpu.sync_copy(col_hbm.at[pl.ds(off, CK)], colb)
        pltpu.sync_copy(ew_hbm.at[pl.ds(off, CK)], ewb)

        @pl.loop(0, CK // L)
        def _(j):
            sl = pl.ds(j * L, L)
            plsc.addupdate_scatter(acc, [colb[sl]], ewb[sl])

    _edge_loop(wid, process)
    pltpu.sync_copy(acc, out_hbm.at[wid])


def _t0_body(d_ref, o_ref):
    o_ref[...] = lax.rsqrt(jnp.sum(d_ref[...], axis=0) + 1.0)


_t0 = pl.pallas_call(
    _t0_body,
    out_shape=jax.ShapeDtypeStruct((N,), jnp.float32),
)


def _t1_body(x_ref, w_ref, dinv_ref, y_ref):
    y = jnp.dot(x_ref[...], w_ref[...], preferred_element_type=jnp.float32)
    y_ref[...] = y * dinv_ref[...]


_t1 = pl.pallas_call(
    _t1_body,
    out_shape=jax.ShapeDtypeStruct((N, D), jnp.float32),
)


def _mid_body(s_ref, y_ref, dinv_ref, b_ref, w_ref, o_ref):
    dinv = dinv_ref[...]
    z = dinv * (s_ref[0] + s_ref[1] + y_ref[...]) + b_ref[...]
    z = jnp.maximum(z, 0.0)
    o_ref[...] = dinv * jnp.dot(z, w_ref[...], preferred_element_type=jnp.float32)


_mid = pl.pallas_call(
    _mid_body,
    out_shape=jax.ShapeDtypeStruct((N, D), jnp.float32),
)


def _head_body(s_ref, y_ref, dinv_ref, b_ref, batch_ref, wl_ref, bl_ref, o_ref):
    z3 = dinv_ref[...] * (s_ref[0] + s_ref[1] + y_ref[...]) + b_ref[...]
    gids = lax.broadcasted_iota(jnp.int32, (N, G), 1)
    onehot = (batch_ref[...] == gids).astype(jnp.float32)
    dn = (((0,), (0,)), ((), ()))
    sums = lax.dot_general(onehot, z3, dn, preferred_element_type=jnp.float32)
    cnt = lax.dot_general(onehot, jnp.ones((N, 1), jnp.float32), dn,
                          preferred_element_type=jnp.float32)
    pooled = sums / jnp.maximum(cnt, 1.0)
    o_ref[...] = jnp.dot(pooled, wl_ref[...],
                         preferred_element_type=jnp.float32) + bl_ref[...]


_head = pl.pallas_call(
    _head_body,
    out_shape=jax.ShapeDtypeStruct((G, D), jnp.float32),
)


def kernel(x, edge_index, batch, edge_attr, W1, b1, W2, b2, W3, b3, Wl, bl):
    row = edge_index[0]
    col = edge_index[1]
    zero_nd = jnp.zeros((N, D), jnp.float32)
    batch2d = batch.reshape(N, 1)
    b1r = b1.reshape(1, D)
    b2r = b2.reshape(1, D)
    b3r = b3.reshape(1, D)
    wl_pad = jnp.pad(Wl, ((0, 0), (0, D - Wl.shape[1])))
    bl_pad = jnp.pad(bl, (0, D - bl.shape[0])).reshape(1, D)

    dpart = _deg_sc(col, edge_attr)
    dinv = _t0(dpart).reshape(N, 1)
    y1 = _t1(x, W1, dinv)
    s1 = _spmm_sc(y1, row, col, edge_attr, zero_nd)
    y2 = _mid(s1, y1, dinv, b1r, W2)
    s2 = _spmm_sc(y2, row, col, edge_attr, zero_nd)
    y3 = _mid(s2, y2, dinv, b2r, W3)
    s3 = _spmm_sc(y3, row, col, edge_attr, zero_nd)
    out = _head(s3, y3, dinv, b3r, batch2d, wl_pad, bl_pad)
    return out[:, :Wl.shape[1]]


# speedup vs baseline: 12.5997x; 1.3533x over previous
import dataclasses
import functools

import jax
import jax.numpy as jnp
from jax import lax
from jax.experimental import pallas as pl
from jax.experimental.pallas import tpu as pltpu
from jax.experimental.pallas import tpu_sc as plsc

N = 10000
E = 320000
D = 128
G = 128

NC = 2
NS = 16
NW = NC * NS
L = 16

EPT = E // NW
SCK = 40
CPT = EPT // SCK

ROWS_A = 624
ROWS_LAST = N - 15 * ROWS_A

_mesh = plsc.VectorSubcoreMesh(core_axis_name="c", subcore_axis_name="s")

_sc_params = pltpu.CompilerParams()
if "needs_layout_passes" in pltpu.CompilerParams.__dataclass_fields__:
    _sc_params = dataclasses.replace(_sc_params, needs_layout_passes=False)
_sc_params_untiled = _sc_params
if "use_tc_tiling_on_sc" in pltpu.CompilerParams.__dataclass_fields__:
    _sc_params_untiled = dataclasses.replace(
        _sc_params, use_tc_tiling_on_sc=False)


def _tile_rows(s, fn):
    @pl.when(s < NS - 1)
    def _():
        fn(s * ROWS_A, ROWS_A)

    @pl.when(s == NS - 1)
    def _():
        fn((NS - 1) * ROWS_A, ROWS_LAST)


@functools.partial(
    pl.kernel,
    out_type=jax.ShapeDtypeStruct((NC, N, D), jnp.float32),
    mesh=_mesh,
    compiler_params=_sc_params_untiled,
    scratch_types=[
        pltpu.VMEM_SHARED((N, D), jnp.float32),
        pltpu.VMEM((EPT,), jnp.int32),
        pltpu.VMEM((CPT, SCK), jnp.int32),
        pltpu.VMEM((2, SCK), jnp.float32),
        pltpu.VMEM((SCK, D // 2), jnp.int32),
        pltpu.VMEM((SCK, D // 2), jnp.int32),
        pltpu.VMEM((SCK, D), jnp.float32),
        pltpu.VMEM((SCK, D), jnp.float32),
        pltpu.SemaphoreType.DMA,
        pltpu.SemaphoreType.DMA,
        pltpu.SemaphoreType.DMA,
        pltpu.SemaphoreType.DMA,
        pltpu.SemaphoreType.DMA,
        pltpu.SemaphoreType.DMA,
    ],
)
def _spmm_sc(y_hbm, row_hbm, col2d_hbm, ew_hbm, zero_hbm, out_hbm,
             acc, rowb, colb, ewb, gb0, gb1, sb0, sb1,
             gs0, gs1, ss0, ss1, es0, es1):
    c = lax.axis_index("c")
    s = lax.axis_index("s")
    wid = s * NC + c
    gbs, sbs = (gb0, gb1), (sb0, sb1)
    gsems, ssems, esems = (gs0, gs1), (ss0, ss1), (es0, es1)

    pltpu.sync_copy(row_hbm.at[pl.ds(wid * EPT, EPT)], rowb)
    pltpu.sync_copy(col2d_hbm.at[wid], colb)
    _tile_rows(s, lambda base, n: pltpu.sync_copy(
        zero_hbm.at[pl.ds(base, n)], acc.at[pl.ds(base, n)]))
    plsc.subcore_barrier()

    def gstart(b, j):
        pltpu.async_copy(y_hbm.at[rowb.at[pl.ds(j * SCK, SCK)]], gbs[b],
                         gsems[b])

    def gwait(b):
        pltpu.make_async_copy(y_hbm.at[rowb.at[pl.ds(0, SCK)]], gbs[b],
                              gsems[b]).wait()

    def sstart(b, j):
        pltpu.async_copy(sbs[b], acc.at[colb.at[j]], ssems[b], add=True)

    def swait(b):
        pltpu.make_async_copy(sbs[b], acc.at[colb.at[0]], ssems[b]).wait()

    def estart(b, j):
        pltpu.async_copy(ew_hbm.at[pl.ds(wid * EPT + j * SCK, SCK)],
                         ewb.at[b], esems[b])

    def ewait(b):
        pltpu.make_async_copy(ew_hbm.at[pl.ds(0, SCK)], ewb.at[b],
                              esems[b]).wait()

    def scale(b):
        iot2 = lax.iota(jnp.int32, L) * 2
        himask = jnp.full((L,), -65536, jnp.int32)

        @pl.loop(0, SCK)
        def _(k):
            wv = plsc.load_gather(
                ewb.at[b], [jnp.full((L,), k, jnp.int32)])
            kf = jnp.full((L,), k, jnp.int32)
            for jj in range(D // (2 * L)):
                xi = gbs[b][k, pl.ds(jj * L, L)]
                ev = plsc.bitcast(xi << 16, jnp.float32)
                od = plsc.bitcast(xi & himask, jnp.float32)
                base = iot2 + (jj * 2 * L)
                plsc.store_scatter(sbs[b], [kf, base], ev * wv)
                plsc.store_scatter(sbs[b], [kf, base + 1], od * wv)

    for b in range(2):
        gstart(b, b)
        estart(b, b)

    MAIN = CPT - (CPT % 2)

    @pl.loop(0, MAIN, step=2)
    def _(j0):
        for b in range(2):
            j = j0 + b
            gwait(b)

            @pl.when(j >= 2)
            def _():
                swait(b)

            ewait(b)
            scale(b)

            @pl.when(j + 2 < CPT)
            def _():
                gstart(b, j + 2)
                estart(b, j + 2)

            sstart(b, j)

    if CPT % 2:
        gwait(0)
        swait(0)
        ewait(0)
        scale(0)
        sstart(0, CPT - 1)
    swait(1)
    swait(0)
    plsc.subcore_barrier()

    _tile_rows(s, lambda base, n: pltpu.sync_copy(
        acc.at[pl.ds(base, n)], out_hbm.at[c, pl.ds(base, n)]))


@functools.partial(
    pl.kernel,
    out_type=jax.ShapeDtypeStruct((NW, N), jnp.float32),
    mesh=_mesh,
    compiler_params=_sc_params,
    scratch_types=[
        pltpu.VMEM((N,), jnp.float32),
        pltpu.VMEM((EPT,), jnp.int32),
        pltpu.VMEM((EPT,), jnp.float32),
    ],
)
def _deg_sc(col_hbm, ew_hbm, out_hbm, acc, colb, ewb):
    c = lax.axis_index("c")
    s = lax.axis_index("s")
    wid = s * NC + c

    @pl.loop(0, N // L)
    def _(i):
        acc[pl.ds(i * L, L)] = jnp.zeros((L,), jnp.float32)

    pltpu.sync_copy(col_hbm.at[pl.ds(wid * EPT, EPT)], colb)
    pltpu.sync_copy(ew_hbm.at[pl.ds(wid * EPT, EPT)], ewb)

    @pl.loop(0, EPT // L)
    def _(j):
        sl = pl.ds(j * L, L)
        plsc.addupdate_scatter(acc, [colb[sl]], ewb[sl])

    pltpu.sync_copy(acc, out_hbm.at[wid])


def _t0_body(d_ref, o_ref):
    o_ref[...] = lax.rsqrt(jnp.sum(d_ref[...], axis=0) + 1.0)


_t0 = pl.pallas_call(
    _t0_body,
    out_shape=jax.ShapeDtypeStruct((N,), jnp.float32),
)


def _t1_body(x_ref, w_ref, dinv_ref, y_ref, ybf_ref):
    y = jnp.dot(x_ref[...], w_ref[...], preferred_element_type=jnp.float32)
    y = y * dinv_ref[...]
    y_ref[...] = y
    ybf_ref[...] = y.astype(jnp.bfloat16)


_t1 = pl.pallas_call(
    _t1_body,
    out_shape=[
        jax.ShapeDtypeStruct((N, D), jnp.float32),
        jax.ShapeDtypeStruct((N, D), jnp.bfloat16),
    ],
)


def _mid_body(s_ref, y_ref, dinv_ref, b_ref, w_ref, o_ref, obf_ref):
    dinv = dinv_ref[...]
    z = dinv * (s_ref[0] + s_ref[1] + y_ref[...]) + b_ref[...]
    z = jnp.maximum(z, 0.0)
    o = dinv * jnp.dot(z, w_ref[...], preferred_element_type=jnp.float32)
    o_ref[...] = o
    obf_ref[...] = o.astype(jnp.bfloat16)


_mid = pl.pallas_call(
    _mid_body,
    out_shape=[
        jax.ShapeDtypeStruct((N, D), jnp.float32),
        jax.ShapeDtypeStruct((N, D), jnp.bfloat16),
    ],
)


def _head_body(s_ref, y_ref, dinv_ref, b_ref, batch_ref, wl_ref, bl_ref, o_ref):
    z3 = dinv_ref[...] * (s_ref[0] + s_ref[1] + y_ref[...]) + b_ref[...]
    gids = lax.broadcasted_iota(jnp.int32, (N, G), 1)
    onehot = (batch_ref[...] == gids).astype(jnp.float32)
    dn = (((0,), (0,)), ((), ()))
    sums = lax.dot_general(onehot, z3, dn, preferred_element_type=jnp.float32)
    cnt = lax.dot_general(onehot, jnp.ones((N, 1), jnp.float32), dn,
                          preferred_element_type=jnp.float32)
    pooled = sums / jnp.maximum(cnt, 1.0)
    o_ref[...] = jnp.dot(pooled, wl_ref[...],
                         preferred_element_type=jnp.float32) + bl_ref[...]


_head = pl.pallas_call(
    _head_body,
    out_shape=jax.ShapeDtypeStruct((G, D), jnp.float32),
)


def kernel(x, edge_index, batch, edge_attr, W1, b1, W2, b2, W3, b3, Wl, bl):
    row = edge_index[0]
    col = edge_index[1]
    zero_nd = jnp.zeros((N, D), jnp.float32)
    batch2d = batch.reshape(N, 1)
    b1r = b1.reshape(1, D)
    b2r = b2.reshape(1, D)
    b3r = b3.reshape(1, D)
    wl_pad = jnp.pad(Wl, ((0, 0), (0, D - Wl.shape[1])))
    bl_pad = jnp.pad(bl, (0, D - bl.shape[0])).reshape(1, D)

    col2d = col.reshape(NW, CPT, SCK)
    def _pack(ybf):
        return lax.bitcast_convert_type(
            ybf.reshape(N, D // 2, 2), jnp.int32)

    dpart = _deg_sc(col, edge_attr)
    dinv = _t0(dpart).reshape(N, 1)
    y1, y1b = _t1(x, W1, dinv)
    s1 = _spmm_sc(_pack(y1b), row, col2d, edge_attr, zero_nd)
    y2, y2b = _mid(s1, y1, dinv, b1r, W2)
    s2 = _spmm_sc(_pack(y2b), row, col2d, edge_attr, zero_nd)
    y3, y3b = _mid(s2, y2, dinv, b2r, W3)
    s3 = _spmm_sc(_pack(y3b), row, col2d, edge_attr, zero_nd)
    out = _head(s3, y3, dinv, b3r, batch2d, wl_pad, bl_pad)
    return out[:, :Wl.shape[1]]


# speedup vs baseline: 12.7311x; 1.0104x over previous
import dataclasses
import functools

import numpy as np

import jax
import jax.numpy as jnp
from jax import lax
from jax.experimental import pallas as pl
from jax.experimental.pallas import tpu as pltpu
from jax.experimental.pallas import tpu_sc as plsc

N = 10000
E = 320000
D = 128
G = 128

NC = 2
NS = 16
NW = NC * NS
L = 16

EPT = E // NW
SCK = 40
CPT = EPT // SCK

ROWS_A = 624
ROWS_LAST = N - 15 * ROWS_A

_sigma = np.empty(D, np.int64)
for _g in range(D // 32):
    _sigma[32 * _g:32 * _g + 16] = 32 * _g + 2 * np.arange(16)
    _sigma[32 * _g + 16:32 * _g + 32] = 32 * _g + 2 * np.arange(16) + 1
_QPERM = np.empty(D, np.int64)
_QPERM[_sigma] = np.arange(D)

_mesh = plsc.VectorSubcoreMesh(core_axis_name="c", subcore_axis_name="s")

_sc_params = pltpu.CompilerParams()
if "needs_layout_passes" in pltpu.CompilerParams.__dataclass_fields__:
    _sc_params = dataclasses.replace(_sc_params, needs_layout_passes=False)
_sc_params_untiled = _sc_params
if "use_tc_tiling_on_sc" in pltpu.CompilerParams.__dataclass_fields__:
    _sc_params_untiled = dataclasses.replace(
        _sc_params, use_tc_tiling_on_sc=False)


def _tile_rows(s, fn):
    @pl.when(s < NS - 1)
    def _():
        fn(s * ROWS_A, ROWS_A)

    @pl.when(s == NS - 1)
    def _():
        fn((NS - 1) * ROWS_A, ROWS_LAST)


@functools.partial(
    pl.kernel,
    out_type=jax.ShapeDtypeStruct((NC, N, D), jnp.float32),
    mesh=_mesh,
    compiler_params=_sc_params_untiled,
    scratch_types=[
        pltpu.VMEM_SHARED((N, D), jnp.float32),
        pltpu.VMEM((EPT,), jnp.int32),
        pltpu.VMEM((CPT, SCK), jnp.int32),
        pltpu.VMEM((2, SCK), jnp.float32),
        pltpu.VMEM((SCK, D // 2), jnp.int32),
        pltpu.VMEM((SCK, D // 2), jnp.int32),
        pltpu.VMEM((SCK, D), jnp.float32),
        pltpu.VMEM((SCK, D), jnp.float32),
        pltpu.SemaphoreType.DMA,
        pltpu.SemaphoreType.DMA,
        pltpu.SemaphoreType.DMA,
        pltpu.SemaphoreType.DMA,
        pltpu.SemaphoreType.DMA,
        pltpu.SemaphoreType.DMA,
    ],
)
def _spmm_sc(y_hbm, row_hbm, col2d_hbm, ew_hbm, zero_hbm, out_hbm,
             acc, rowb, colb, ewb, gb0, gb1, sb0, sb1,
             gs0, gs1, ss0, ss1, es0, es1):
    c = lax.axis_index("c")
    s = lax.axis_index("s")
    wid = s * NC + c
    gbs, sbs = (gb0, gb1), (sb0, sb1)
    gsems, ssems, esems = (gs0, gs1), (ss0, ss1), (es0, es1)

    pltpu.sync_copy(row_hbm.at[pl.ds(wid * EPT, EPT)], rowb)
    pltpu.sync_copy(col2d_hbm.at[wid], colb)
    _tile_rows(s, lambda base, n: pltpu.sync_copy(
        zero_hbm.at[pl.ds(base, n)], acc.at[pl.ds(base, n)]))
    plsc.subcore_barrier()

    def gstart(b, j):
        pltpu.async_copy(y_hbm.at[rowb.at[pl.ds(j * SCK, SCK)]], gbs[b],
                         gsems[b])

    def gwait(b):
        pltpu.make_async_copy(y_hbm.at[rowb.at[pl.ds(0, SCK)]], gbs[b],
                              gsems[b]).wait()

    def sstart(b, j):
        pltpu.async_copy(sbs[b], acc.at[colb.at[j]], ssems[b], add=True)

    def swait(b):
        pltpu.make_async_copy(sbs[b], acc.at[colb.at[0]], ssems[b]).wait()

    def estart(b, j):
        pltpu.async_copy(ew_hbm.at[pl.ds(wid * EPT + j * SCK, SCK)],
                         ewb.at[b], esems[b])

    def ewait(b):
        pltpu.make_async_copy(ew_hbm.at[pl.ds(0, SCK)], ewb.at[b],
                              esems[b]).wait()

    def scale(b):
        himask = jnp.full((L,), -65536, jnp.int32)

        @pl.loop(0, SCK, step=2)
        def _(k0):
            for du in range(2):
                k = k0 + du
                wv = plsc.load_gather(
                    ewb.at[b], [jnp.full((L,), k, jnp.int32)])
                for jj in range(D // (2 * L)):
                    xi = gbs[b][k, pl.ds(jj * L, L)]
                    ev = plsc.bitcast(xi << 16, jnp.float32)
                    od = plsc.bitcast(xi & himask, jnp.float32)
                    sbs[b][k, pl.ds(jj * 2 * L, L)] = ev * wv
                    sbs[b][k, pl.ds(jj * 2 * L + L, L)] = od * wv

    for b in range(2):
        gstart(b, b)
        estart(b, b)

    MAIN = CPT - (CPT % 2)

    @pl.loop(0, MAIN, step=2)
    def _(j0):
        for b in range(2):
            j = j0 + b
            gwait(b)

            @pl.when(j >= 2)
            def _():
                swait(b)

            ewait(b)
            scale(b)

            @pl.when(j + 2 < CPT)
            def _():
                gstart(b, j + 2)
                estart(b, j + 2)

            sstart(b, j)

    if CPT % 2:
        gwait(0)
        swait(0)
        ewait(0)
        scale(0)
        sstart(0, CPT - 1)
    swait(1)
    swait(0)
    plsc.subcore_barrier()

    _tile_rows(s, lambda base, n: pltpu.sync_copy(
        acc.at[pl.ds(base, n)], out_hbm.at[c, pl.ds(base, n)]))


@functools.partial(
    pl.kernel,
    out_type=jax.ShapeDtypeStruct((NW, N), jnp.float32),
    mesh=_mesh,
    compiler_params=_sc_params,
    scratch_types=[
        pltpu.VMEM((N,), jnp.float32),
        pltpu.VMEM((EPT,), jnp.int32),
        pltpu.VMEM((EPT,), jnp.float32),
    ],
)
def _deg_sc(col_hbm, ew_hbm, out_hbm, acc, colb, ewb):
    c = lax.axis_index("c")
    s = lax.axis_index("s")
    wid = s * NC + c

    @pl.loop(0, N // L)
    def _(i):
        acc[pl.ds(i * L, L)] = jnp.zeros((L,), jnp.float32)

    pltpu.sync_copy(col_hbm.at[pl.ds(wid * EPT, EPT)], colb)
    pltpu.sync_copy(ew_hbm.at[pl.ds(wid * EPT, EPT)], ewb)

    @pl.loop(0, EPT // L)
    def _(j):
        sl = pl.ds(j * L, L)
        plsc.addupdate_scatter(acc, [colb[sl]], ewb[sl])

    pltpu.sync_copy(acc, out_hbm.at[wid])


def _t0_body(d_ref, o_ref):
    o_ref[...] = lax.rsqrt(jnp.sum(d_ref[...], axis=0) + 1.0)


_t0 = pl.pallas_call(
    _t0_body,
    out_shape=jax.ShapeDtypeStruct((N,), jnp.float32),
)


def _t1_body(x_ref, w_ref, wq_ref, dinv_ref, y_ref, ybf_ref):
    x = x_ref[...]
    dinv = dinv_ref[...]
    y_ref[...] = dinv * jnp.dot(x, w_ref[...],
                                preferred_element_type=jnp.float32)
    yq = dinv * jnp.dot(x, wq_ref[...], preferred_element_type=jnp.float32)
    ybf_ref[...] = yq.astype(jnp.bfloat16)


_t1 = pl.pallas_call(
    _t1_body,
    out_shape=[
        jax.ShapeDtypeStruct((N, D), jnp.float32),
        jax.ShapeDtypeStruct((N, D), jnp.bfloat16),
    ],
)


def _mid_body(s_ref, y_ref, dinv_ref, b_ref, w_ref, wq_ref, o_ref, obf_ref):
    dinv = dinv_ref[...]
    z = dinv * (s_ref[0] + s_ref[1] + y_ref[...]) + b_ref[...]
    z = jnp.maximum(z, 0.0)
    o_ref[...] = dinv * jnp.dot(z, w_ref[...],
                                preferred_element_type=jnp.float32)
    oq = dinv * jnp.dot(z, wq_ref[...], preferred_element_type=jnp.float32)
    obf_ref[...] = oq.astype(jnp.bfloat16)


_mid = pl.pallas_call(
    _mid_body,
    out_shape=[
        jax.ShapeDtypeStruct((N, D), jnp.float32),
        jax.ShapeDtypeStruct((N, D), jnp.bfloat16),
    ],
)


def _head_body(s_ref, y_ref, dinv_ref, b_ref, batch_ref, wl_ref, bl_ref, o_ref):
    z3 = dinv_ref[...] * (s_ref[0] + s_ref[1] + y_ref[...]) + b_ref[...]
    gids = lax.broadcasted_iota(jnp.int32, (N, G), 1)
    onehot = (batch_ref[...] == gids).astype(jnp.float32)
    dn = (((0,), (0,)), ((), ()))
    sums = lax.dot_general(onehot, z3, dn, preferred_element_type=jnp.float32)
    cnt = lax.dot_general(onehot, jnp.ones((N, 1), jnp.float32), dn,
                          preferred_element_type=jnp.float32)
    pooled = sums / jnp.maximum(cnt, 1.0)
    o_ref[...] = jnp.dot(pooled, wl_ref[...],
                         preferred_element_type=jnp.float32) + bl_ref[...]


_head = pl.pallas_call(
    _head_body,
    out_shape=jax.ShapeDtypeStruct((G, D), jnp.float32),
)


def kernel(x, edge_index, batch, edge_attr, W1, b1, W2, b2, W3, b3, Wl, bl):
    row = edge_index[0]
    col = edge_index[1]
    zero_nd = jnp.zeros((N, D), jnp.float32)
    batch2d = batch.reshape(N, 1)
    b1r = b1.reshape(1, D)
    b2r = b2.reshape(1, D)
    b3r = b3.reshape(1, D)
    wl_pad = jnp.pad(Wl, ((0, 0), (0, D - Wl.shape[1])))
    bl_pad = jnp.pad(bl, (0, D - bl.shape[0])).reshape(1, D)

    col2d = col.reshape(NW, CPT, SCK)
    def _pack(ybf):
        return lax.bitcast_convert_type(
            ybf.reshape(N, D // 2, 2), jnp.int32)

    qperm = jnp.asarray(_QPERM)
    W1q, W2q, W3q = W1[:, qperm], W2[:, qperm], W3[:, qperm]
    dpart = _deg_sc(col, edge_attr)
    dinv = _t0(dpart).reshape(N, 1)
    y1, y1b = _t1(x, W1, W1q, dinv)
    s1 = _spmm_sc(_pack(y1b), row, col2d, edge_attr, zero_nd)
    y2, y2b = _mid(s1, y1, dinv, b1r, W2, W2q)
    s2 = _spmm_sc(_pack(y2b), row, col2d, edge_attr, zero_nd)
    y3, y3b = _mid(s2, y2, dinv, b2r, W3, W3q)
    s3 = _spmm_sc(_pack(y3b), row, col2d, edge_attr, zero_nd)
    out = _head(s3, y3, dinv, b3r, batch2d, wl_pad, bl_pad)
    return out[:, :Wl.shape[1]]
